# baseline (speedup 1.0000x reference)
import functools

import jax
import jax.numpy as jnp
from jax import lax
from jax.experimental import pallas as pl
from jax.experimental.pallas import tpu as pltpu
from jax.experimental.pallas import tpu_sc as plsc

NN = 100000
EE = 1600000
FIN = 128
HH = 32
HHH = 16
OUTD = 16
GG = 64

NC = 2
NS = 16
ROW = 128
RPC = 4
CHUNK = ROW * RPC

E_ROWS = EE // ROW
E_CHUNKS = E_ROWS // RPC
SCT_Q = E_CHUNKS // NS
SCT_R = E_CHUNKS - SCT_Q * NS
DEG_Q = E_CHUNKS // (NC * NS)
DEG_R = E_CHUNKS - DEG_Q * NC * NS

NACC = 100096
NPT = NACC // NS
ZCH = 368
NP = 100352
NPP = NP // NS

BLK = 4000

_mesh = plsc.VectorSubcoreMesh(
    core_axis_name="c", subcore_axis_name="s", num_cores=NC, num_subcores=NS)


@functools.partial(
    pl.kernel,
    out_type=jax.ShapeDtypeStruct((2 * NP,), jnp.float32),
    mesh=_mesh,
    compiler_params=pltpu.CompilerParams(use_tc_tiling_on_sc=False),
    scratch_types=[
        pltpu.VMEM_SHARED((NP,), jnp.float32),
        pltpu.VMEM((RPC, ROW), jnp.int32),
        pltpu.VMEM((ROW,), jnp.float32),
        pltpu.VMEM((NPP,), jnp.float32),
    ],
)
def _deg_kernel(dst_hbm, out_hbm, acc, dbuf, ones_v, zbuf):
    c = lax.axis_index("c")
    s = lax.axis_index("s")
    w = c * NS + s

    def zfill(i, carry):
        zbuf[pl.ds(pl.multiple_of(i * 16, 16), 16)] = jnp.zeros(
            (16,), jnp.float32)
        return carry

    lax.fori_loop(0, NPP // 16, zfill, 0)
    pltpu.sync_copy(zbuf, acc.at[pl.ds(s * NPP, NPP)])
    for i in range(ROW // 16):
        ones_v[pl.ds(i * 16, 16)] = jnp.ones((16,), jnp.float32)
    plsc.subcore_barrier()

    ck0 = DEG_Q * w + jnp.minimum(w, DEG_R)
    nck = DEG_Q + jnp.where(w < DEG_R, 1, 0)

    def body(i, carry):
        row0 = (ck0 + i) * RPC
        pltpu.sync_copy(dst_hbm.at[pl.ds(row0, RPC)], dbuf)
        for j in range(RPC):
            pltpu.sync_copy(ones_v, acc.at[dbuf.at[j]], add=True)
        return carry

    lax.fori_loop(0, nck, body, 0)
    plsc.subcore_barrier()
    pltpu.sync_copy(acc.at[pl.ds(s * NPP, NPP)], zbuf)
    pltpu.sync_copy(zbuf, out_hbm.at[pl.ds(c * NP + s * NPP, NPP)])


@functools.partial(
    pl.kernel,
    out_type=jax.ShapeDtypeStruct((2 * NACC, HHH), jnp.float32),
    mesh=_mesh,
    compiler_params=pltpu.CompilerParams(use_tc_tiling_on_sc=False),
    scratch_types=[
        pltpu.VMEM_SHARED((NACC, HHH), jnp.float32),
        pltpu.VMEM((RPC, ROW), jnp.int32),
        pltpu.VMEM((RPC, ROW), jnp.int32),
        pltpu.VMEM((CHUNK, HHH), jnp.float32),
        pltpu.VMEM((ZCH, HHH), jnp.float32),
        pltpu.SemaphoreType.DMA,
    ],
)
def _scatter_kernel(g_hbm, src_hbm, dst_hbm, out_hbm, acc, sbuf, dbuf, rows,
                    zbuf, gsem):
    c = lax.axis_index("c")
    s = lax.axis_index("s")

    def zfill(i, carry):
        zbuf[i] = jnp.zeros((HHH,), jnp.float32)
        return carry

    lax.fori_loop(0, ZCH, zfill, 0)
    for k in range(NPT // ZCH):
        pltpu.sync_copy(zbuf, acc.at[pl.ds(s * NPT + k * ZCH, ZCH)])

    plsc.subcore_barrier()

    off = c * NN
    ck0 = SCT_Q * s + jnp.minimum(s, SCT_R)
    nck = SCT_Q + jnp.where(s < SCT_R, 1, 0)

    def body(i, carry):
        row0 = (ck0 + i) * RPC
        pltpu.sync_copy(src_hbm.at[pl.ds(row0, RPC)], sbuf)
        pltpu.sync_copy(dst_hbm.at[pl.ds(row0, RPC)], dbuf)
        for j in range(RPC):
            for k in range(ROW // 16):
                sbuf[j, pl.ds(k * 16, 16)] = sbuf[j, pl.ds(k * 16, 16)] + off
        cps = [
            pltpu.async_copy(g_hbm.at[sbuf.at[j]],
                             rows.at[pl.ds(j * ROW, ROW)], gsem)
            for j in range(RPC)
        ]
        for j in range(RPC):
            cps[j].wait()
            pltpu.sync_copy(rows.at[pl.ds(j * ROW, ROW)],
                            acc.at[dbuf.at[j]], add=True)
        return carry

    lax.fori_loop(0, nck, body, 0)
    plsc.subcore_barrier()
    for k in range(NPT // ZCH):
        pltpu.sync_copy(acc.at[pl.ds(s * NPT + k * ZCH, ZCH)], zbuf)
        pltpu.sync_copy(zbuf, out_hbm.at[pl.ds(c * NACC + s * NPT + k * ZCH,
                                               ZCH)])


def _tc0_body(x_ref, cnt_ref, w_ref, g_ref, dinv_ref):
    deg = cnt_ref[0] + cnt_ref[1] + 1.0
    dinv = lax.rsqrt(deg)
    dinv_ref[...] = dinv
    h = jnp.dot(x_ref[...], w_ref[...], preferred_element_type=jnp.float32)
    g = h * dinv
    g_ref[0] = g[:, :HHH]
    g_ref[1] = g[:, HHH:]


def _tc0(x, cnt3, w0):
    return pl.pallas_call(
        _tc0_body,
        grid=(NN // BLK,),
        in_specs=[
            pl.BlockSpec((BLK, FIN), lambda i: (i, 0)),
            pl.BlockSpec((2, BLK, 1), lambda i: (0, i, 0)),
            pl.BlockSpec((FIN, HH), lambda i: (0, 0)),
        ],
        out_specs=[
            pl.BlockSpec((2, BLK, HHH), lambda i: (0, i, 0)),
            pl.BlockSpec((BLK, 1), lambda i: (i, 0)),
        ],
        out_shape=[
            jax.ShapeDtypeStruct((2, NN, HHH), jnp.float32),
            jax.ShapeDtypeStruct((NN, 1), jnp.float32),
        ],
    )(x, cnt3, w0)


def _tcmid_body(agg_ref, g_ref, dinv_ref, b_ref, w_ref, gout_ref):
    dinv = dinv_ref[...]
    ssum = jnp.concatenate([agg_ref[0] + g_ref[0], agg_ref[1] + g_ref[1]],
                           axis=1)
    o = ssum * dinv + b_ref[...]
    r = jnp.maximum(o, 0.0)
    h = jnp.dot(r, w_ref[...], preferred_element_type=jnp.float32)
    g = h * dinv
    gout_ref[0] = g[:, :HHH]
    gout_ref[1] = g[:, HHH:]


def _tcmid(agg3, g3, dinv, b, w):
    spec = pl.BlockSpec((2, BLK, HHH), lambda i: (0, i, 0))
    return pl.pallas_call(
        _tcmid_body,
        grid=(NN // BLK,),
        in_specs=[
            spec,
            spec,
            pl.BlockSpec((BLK, 1), lambda i: (i, 0)),
            pl.BlockSpec((1, HH), lambda i: (0, 0)),
            pl.BlockSpec((HH, HH), lambda i: (0, 0)),
        ],
        out_specs=spec,
        out_shape=jax.ShapeDtypeStruct((2, NN, HHH), jnp.float32),
    )(agg3, g3, dinv, b, w)


def _tcpool_body(agg_ref, g_ref, dinv_ref, b_ref, batch_ref, wl_ref, bl_ref,
                 out_ref, acc_ref):
    i = pl.program_id(0)

    @pl.when(i == 0)
    def _():
        acc_ref[...] = jnp.zeros_like(acc_ref)

    ssum = jnp.concatenate([agg_ref[0] + g_ref[0], agg_ref[1] + g_ref[1]],
                           axis=1)
    h = ssum * dinv_ref[...] + b_ref[...]
    hext = jnp.concatenate([h, jnp.ones((BLK, 1), jnp.float32)], axis=1)
    ids = lax.broadcasted_iota(jnp.int32, (1, GG), 1)
    oh = (batch_ref[...] == ids).astype(jnp.float32)
    acc_ref[...] += lax.dot_general(
        oh, hext, (((0,), (0,)), ((), ())),
        preferred_element_type=jnp.float32)

    @pl.when(i == NN // BLK - 1)
    def _():
        sums = acc_ref[:, :HH]
        cnt = jnp.maximum(acc_ref[:, HH:HH + 1], 1.0)
        pooled = sums / cnt
        out_ref[...] = jnp.dot(
            pooled, wl_ref[...],
            preferred_element_type=jnp.float32) + bl_ref[...]


def _tcpool(agg3, g3, dinv, b, batch2, wl, bl):
    spec = pl.BlockSpec((2, BLK, HHH), lambda i: (0, i, 0))
    return pl.pallas_call(
        _tcpool_body,
        grid=(NN // BLK,),
        in_specs=[
            spec,
            spec,
            pl.BlockSpec((BLK, 1), lambda i: (i, 0)),
            pl.BlockSpec((1, HH), lambda i: (0, 0)),
            pl.BlockSpec((BLK, 1), lambda i: (i, 0)),
            pl.BlockSpec((HH, OUTD), lambda i: (0, 0)),
            pl.BlockSpec((1, OUTD), lambda i: (0, 0)),
        ],
        out_specs=pl.BlockSpec((GG, OUTD), lambda i: (0, 0)),
        out_shape=jax.ShapeDtypeStruct((GG, OUTD), jnp.float32),
        scratch_shapes=[pltpu.VMEM((GG, HH + 1), jnp.float32)],
    )(agg3, g3, dinv, b, batch2, wl, bl)


def kernel(x, edge_index, batch, W0, b0, W1, b1, W2, b2, Wl, bl):
    src2 = edge_index[0].reshape(E_ROWS, ROW)
    dst3 = edge_index[1].reshape(E_ROWS, ROW)

    cnt = _deg_kernel(dst3)
    cnt3 = cnt.reshape(2, NP, 1)

    g3, dinv = _tc0(x, cnt3, W0)

    agg = _scatter_kernel(g3.reshape(2 * NN, HHH), src2, dst3)
    g3 = _tcmid(agg.reshape(2, NACC, HHH), g3, dinv, b0.reshape(1, HH), W1)

    agg = _scatter_kernel(g3.reshape(2 * NN, HHH), src2, dst3)
    g3 = _tcmid(agg.reshape(2, NACC, HHH), g3, dinv, b1.reshape(1, HH), W2)

    agg = _scatter_kernel(g3.reshape(2 * NN, HHH), src2, dst3)
    return _tcpool(agg.reshape(2, NACC, HHH), g3, dinv, b2.reshape(1, HH),
                   batch.reshape(NN, 1), Wl, bl.reshape(1, OUTD))


# speedup vs baseline: 1.2011x; 1.2011x over previous
import functools

import jax
import jax.numpy as jnp
from jax import lax
from jax.experimental import pallas as pl
from jax.experimental.pallas import tpu as pltpu
from jax.experimental.pallas import tpu_sc as plsc

NN = 100000
EE = 1600000
FIN = 128
HH = 32
HHH = 16
OUTD = 16
GG = 64

NC = 2
NS = 16
ROW = 128
RPC = 8
CHUNK = ROW * RPC

E_ROWS = 12544
E_PAD = E_ROWS * ROW
ROWS_PER_TILE = E_ROWS // NS
CHUNKS_PER_TILE = ROWS_PER_TILE // RPC
DEG_ROWS_PER_W = E_ROWS // (NC * NS)

NACC = 100096
NPT = NACC // NS
ZCH = 368
NP = 100352
NPP = NP // NS

BLK = 4000

_mesh = plsc.VectorSubcoreMesh(
    core_axis_name="c", subcore_axis_name="s", num_cores=NC, num_subcores=NS)


@functools.partial(
    pl.kernel,
    out_type=jax.ShapeDtypeStruct((2 * NP,), jnp.float32),
    mesh=_mesh,
    compiler_params=pltpu.CompilerParams(use_tc_tiling_on_sc=False),
    scratch_types=[
        pltpu.VMEM_SHARED((NP,), jnp.float32),
        pltpu.VMEM((RPC, ROW), jnp.int32),
        pltpu.VMEM((ROW,), jnp.float32),
        pltpu.VMEM((NPP,), jnp.float32),
    ],
)
def _deg_kernel(dst_hbm, out_hbm, acc, dbuf, ones_v, zbuf):
    c = lax.axis_index("c")
    s = lax.axis_index("s")
    w = c * NS + s

    def zfill(i, carry):
        zbuf[pl.ds(pl.multiple_of(i * 16, 16), 16)] = jnp.zeros(
            (16,), jnp.float32)
        return carry

    lax.fori_loop(0, NPP // 16, zfill, 0)
    pltpu.sync_copy(zbuf, acc.at[pl.ds(s * NPP, NPP)])
    for i in range(ROW // 16):
        ones_v[pl.ds(i * 16, 16)] = jnp.ones((16,), jnp.float32)
    plsc.subcore_barrier()

    def body(i, carry):
        row0 = w * DEG_ROWS_PER_W + i * RPC
        pltpu.sync_copy(dst_hbm.at[pl.ds(row0, RPC)], dbuf)
        for j in range(RPC):
            pltpu.sync_copy(ones_v, acc.at[dbuf.at[j]], add=True)
        return carry

    lax.fori_loop(0, DEG_ROWS_PER_W // RPC, body, 0)
    plsc.subcore_barrier()
    pltpu.sync_copy(acc.at[pl.ds(s * NPP, NPP)], zbuf)
    pltpu.sync_copy(zbuf, out_hbm.at[pl.ds(c * NP + s * NPP, NPP)])


@functools.partial(
    pl.kernel,
    out_type=jax.ShapeDtypeStruct((2 * NACC, HHH), jnp.float32),
    mesh=_mesh,
    compiler_params=pltpu.CompilerParams(use_tc_tiling_on_sc=False),
    scratch_types=[
        pltpu.VMEM_SHARED((NACC, HHH), jnp.float32),
        pltpu.VMEM((RPC, ROW), jnp.int32),
        pltpu.VMEM((RPC, ROW), jnp.int32),
        pltpu.VMEM((CHUNK, HHH), jnp.float32),
        pltpu.VMEM((ZCH, HHH), jnp.float32),
        pltpu.SemaphoreType.DMA,
    ],
)
def _scatter_kernel(g_hbm, src_hbm, dst_hbm, out_hbm, acc, sbuf, dbuf, rows,
                    zbuf, gsem):
    c = lax.axis_index("c")
    s = lax.axis_index("s")

    def zfill(i, carry):
        zbuf[i] = jnp.zeros((HHH,), jnp.float32)
        return carry

    lax.fori_loop(0, ZCH, zfill, 0)
    for k in range(NPT // ZCH):
        pltpu.sync_copy(zbuf, acc.at[pl.ds(s * NPT + k * ZCH, ZCH)])

    plsc.subcore_barrier()

    off = c * NN

    def body(i, carry):
        row0 = s * ROWS_PER_TILE + i * RPC
        pltpu.sync_copy(src_hbm.at[pl.ds(row0, RPC)], sbuf)
        pltpu.sync_copy(dst_hbm.at[pl.ds(row0, RPC)], dbuf)
        for j in range(RPC):
            for k in range(ROW // 16):
                sbuf[j, pl.ds(k * 16, 16)] = sbuf[j, pl.ds(k * 16, 16)] + off
        cps = [
            pltpu.async_copy(g_hbm.at[sbuf.at[j]],
                             rows.at[pl.ds(j * ROW, ROW)], gsem)
            for j in range(RPC)
        ]
        for j in range(RPC):
            cps[j].wait()
            pltpu.sync_copy(rows.at[pl.ds(j * ROW, ROW)],
                            acc.at[dbuf.at[j]], add=True)
        return carry

    lax.fori_loop(0, CHUNKS_PER_TILE, body, 0)
    plsc.subcore_barrier()
    for k in range(NPT // ZCH):
        pltpu.sync_copy(acc.at[pl.ds(s * NPT + k * ZCH, ZCH)], zbuf)
        pltpu.sync_copy(zbuf, out_hbm.at[pl.ds(c * NACC + s * NPT + k * ZCH,
                                               ZCH)])


def _tc0_body(x_ref, cnt_ref, w_ref, g_ref, dinv_ref):
    deg = cnt_ref[0] + cnt_ref[1] + 1.0
    dinv = lax.rsqrt(deg)
    dinv_ref[...] = dinv
    h = jnp.dot(x_ref[...], w_ref[...], preferred_element_type=jnp.float32)
    g = h * dinv
    g_ref[0] = g[:, :HHH]
    g_ref[1] = g[:, HHH:]


def _tc0(x, cnt3, w0):
    return pl.pallas_call(
        _tc0_body,
        grid=(NN // BLK,),
        in_specs=[
            pl.BlockSpec((BLK, FIN), lambda i: (i, 0)),
            pl.BlockSpec((2, BLK, 1), lambda i: (0, i, 0)),
            pl.BlockSpec((FIN, HH), lambda i: (0, 0)),
        ],
        out_specs=[
            pl.BlockSpec((2, BLK, HHH), lambda i: (0, i, 0)),
            pl.BlockSpec((BLK, 1), lambda i: (i, 0)),
        ],
        out_shape=[
            jax.ShapeDtypeStruct((2, NN, HHH), jnp.float32),
            jax.ShapeDtypeStruct((NN, 1), jnp.float32),
        ],
    )(x, cnt3, w0)


def _tcmid_body(agg_ref, g_ref, dinv_ref, b_ref, w_ref, gout_ref):
    dinv = dinv_ref[...]
    ssum = jnp.concatenate([agg_ref[0] + g_ref[0], agg_ref[1] + g_ref[1]],
                           axis=1)
    o = ssum * dinv + b_ref[...]
    r = jnp.maximum(o, 0.0)
    h = jnp.dot(r, w_ref[...], preferred_element_type=jnp.float32)
    g = h * dinv
    gout_ref[0] = g[:, :HHH]
    gout_ref[1] = g[:, HHH:]


def _tcmid(agg3, g3, dinv, b, w):
    spec = pl.BlockSpec((2, BLK, HHH), lambda i: (0, i, 0))
    return pl.pallas_call(
        _tcmid_body,
        grid=(NN // BLK,),
        in_specs=[
            spec,
            spec,
            pl.BlockSpec((BLK, 1), lambda i: (i, 0)),
            pl.BlockSpec((1, HH), lambda i: (0, 0)),
            pl.BlockSpec((HH, HH), lambda i: (0, 0)),
        ],
        out_specs=spec,
        out_shape=jax.ShapeDtypeStruct((2, NN, HHH), jnp.float32),
    )(agg3, g3, dinv, b, w)


def _tcpool_body(agg_ref, g_ref, dinv_ref, b_ref, batch_ref, wl_ref, bl_ref,
                 out_ref, acc_ref):
    i = pl.program_id(0)

    @pl.when(i == 0)
    def _():
        acc_ref[...] = jnp.zeros_like(acc_ref)

    ssum = jnp.concatenate([agg_ref[0] + g_ref[0], agg_ref[1] + g_ref[1]],
                           axis=1)
    h = ssum * dinv_ref[...] + b_ref[...]
    hext = jnp.concatenate([h, jnp.ones((BLK, 1), jnp.float32)], axis=1)
    ids = lax.broadcasted_iota(jnp.int32, (1, GG), 1)
    oh = (batch_ref[...] == ids).astype(jnp.float32)
    acc_ref[...] += lax.dot_general(
        oh, hext, (((0,), (0,)), ((), ())),
        preferred_element_type=jnp.float32)

    @pl.when(i == NN // BLK - 1)
    def _():
        sums = acc_ref[:, :HH]
        cnt = jnp.maximum(acc_ref[:, HH:HH + 1], 1.0)
        pooled = sums / cnt
        out_ref[...] = jnp.dot(
            pooled, wl_ref[...],
            preferred_element_type=jnp.float32) + bl_ref[...]


def _tcpool(agg3, g3, dinv, b, batch2, wl, bl):
    spec = pl.BlockSpec((2, BLK, HHH), lambda i: (0, i, 0))
    return pl.pallas_call(
        _tcpool_body,
        grid=(NN // BLK,),
        in_specs=[
            spec,
            spec,
            pl.BlockSpec((BLK, 1), lambda i: (i, 0)),
            pl.BlockSpec((1, HH), lambda i: (0, 0)),
            pl.BlockSpec((BLK, 1), lambda i: (i, 0)),
            pl.BlockSpec((HH, OUTD), lambda i: (0, 0)),
            pl.BlockSpec((1, OUTD), lambda i: (0, 0)),
        ],
        out_specs=pl.BlockSpec((GG, OUTD), lambda i: (0, 0)),
        out_shape=jax.ShapeDtypeStruct((GG, OUTD), jnp.float32),
        scratch_shapes=[pltpu.VMEM((GG, HH + 1), jnp.float32)],
    )(agg3, g3, dinv, b, batch2, wl, bl)


def kernel(x, edge_index, batch, W0, b0, W1, b1, W2, b2, Wl, bl):
    npad = E_PAD - EE
    src2 = jnp.concatenate(
        [edge_index[0], jnp.zeros((npad,), jnp.int32)]).reshape(E_ROWS, ROW)
    dst3 = jnp.concatenate(
        [edge_index[1], jnp.full((npad,), NN, jnp.int32)]).reshape(E_ROWS, ROW)

    cnt = _deg_kernel(dst3)
    cnt3 = cnt.reshape(2, NP, 1)

    g3, dinv = _tc0(x, cnt3, W0)

    agg = _scatter_kernel(g3.reshape(2 * NN, HHH), src2, dst3)
    g3 = _tcmid(agg.reshape(2, NACC, HHH), g3, dinv, b0.reshape(1, HH), W1)

    agg = _scatter_kernel(g3.reshape(2 * NN, HHH), src2, dst3)
    g3 = _tcmid(agg.reshape(2, NACC, HHH), g3, dinv, b1.reshape(1, HH), W2)

    agg = _scatter_kernel(g3.reshape(2 * NN, HHH), src2, dst3)
    return _tcpool(agg.reshape(2, NACC, HHH), g3, dinv, b2.reshape(1, HH),
                   batch.reshape(NN, 1), Wl, bl.reshape(1, OUTD))


# speedup vs baseline: 1.7903x; 1.4905x over previous
import functools

import jax
import jax.numpy as jnp
from jax import lax
from jax.experimental import pallas as pl
from jax.experimental.pallas import tpu as pltpu
from jax.experimental.pallas import tpu_sc as plsc

NN = 100000
EE = 1600000
FIN = 128
HH = 32
HHH = 16
OUTD = 16
GG = 64
GP = 128

NC = 2
NS = 16
ROW = 128
RPC = 8
CHUNK = ROW * RPC

E_ROWS = 12544
E_PAD = E_ROWS * ROW
ROWS_PER_TILE = E_ROWS // NS
CHUNKS_PER_TILE = ROWS_PER_TILE // RPC
DEG_ROWS_PER_W = E_ROWS // (NC * NS)

NP = 100352
NPP = NP // NS
NPK = NP // 8
ZCH = 392
NPT_CHUNKS = NPP // ZCH

B_ROWS = NP // ROW
B_PT = B_ROWS // NS
B_RPC = 7

BLK8 = 784
GRID = NPK // BLK8

_mesh = plsc.VectorSubcoreMesh(
    core_axis_name="c", subcore_axis_name="s", num_cores=NC, num_subcores=NS)


@functools.partial(
    pl.kernel,
    out_type=jax.ShapeDtypeStruct((NC * NP, HHH), jnp.float32),
    mesh=_mesh,
    compiler_params=pltpu.CompilerParams(use_tc_tiling_on_sc=False),
    scratch_types=[
        pltpu.VMEM_SHARED((NP, HHH), jnp.float32),
        pltpu.VMEM((RPC, ROW), jnp.int32),
        pltpu.VMEM((ROW, HHH), jnp.float32),
        pltpu.VMEM((ZCH, HHH), jnp.float32),
    ],
)
def _deg_kernel(dst_hbm, out_hbm, acc, dbuf, onesb, zbuf):
    c = lax.axis_index("c")
    s = lax.axis_index("s")
    w = c * NS + s

    def zfill(i, carry):
        zbuf[i] = jnp.zeros((HHH,), jnp.float32)
        return carry

    lax.fori_loop(0, ZCH, zfill, 0)
    for k in range(NPT_CHUNKS):
        pltpu.sync_copy(zbuf, acc.at[pl.ds(s * NPP + k * ZCH, ZCH)])

    def ofill(i, carry):
        onesb[i] = jnp.ones((HHH,), jnp.float32)
        return carry

    lax.fori_loop(0, ROW, ofill, 0)
    plsc.subcore_barrier()

    def body(i, carry):
        row0 = w * DEG_ROWS_PER_W + i * RPC
        pltpu.sync_copy(dst_hbm.at[pl.ds(row0, RPC)], dbuf)
        for j in range(RPC):
            pltpu.sync_copy(onesb, acc.at[dbuf.at[j]], add=True)
        return carry

    lax.fori_loop(0, DEG_ROWS_PER_W // RPC, body, 0)
    plsc.subcore_barrier()
    for k in range(NPT_CHUNKS):
        pltpu.sync_copy(acc.at[pl.ds(s * NPP + k * ZCH, ZCH)], zbuf)
        pltpu.sync_copy(zbuf, out_hbm.at[pl.ds(c * NP + s * NPP + k * ZCH,
                                               ZCH)])


@functools.partial(
    pl.kernel,
    out_type=jax.ShapeDtypeStruct((NC * NP, HHH), jnp.float32),
    mesh=_mesh,
    compiler_params=pltpu.CompilerParams(use_tc_tiling_on_sc=False),
    scratch_types=[
        pltpu.VMEM_SHARED((NP, HHH), jnp.float32),
        pltpu.VMEM((RPC, ROW), jnp.int32),
        pltpu.VMEM((RPC, ROW), jnp.int32),
        pltpu.VMEM((CHUNK, HHH), jnp.float32),
        pltpu.VMEM((ZCH, HHH), jnp.float32),
        pltpu.SemaphoreType.DMA,
    ],
)
def _scatter_kernel(ga_hbm, gb_hbm, src_hbm, dst_hbm, out_hbm, acc, sbuf,
                    dbuf, rows, zbuf, gsem):
    c = lax.axis_index("c")
    s = lax.axis_index("s")

    def zfill(i, carry):
        zbuf[i] = jnp.zeros((HHH,), jnp.float32)
        return carry

    lax.fori_loop(0, ZCH, zfill, 0)
    for k in range(NPT_CHUNKS):
        pltpu.sync_copy(zbuf, acc.at[pl.ds(s * NPP + k * ZCH, ZCH)])

    plsc.subcore_barrier()

    def walk(gref):
        def body(i, carry):
            row0 = s * ROWS_PER_TILE + i * RPC
            pltpu.sync_copy(src_hbm.at[pl.ds(row0, RPC)], sbuf)
            pltpu.sync_copy(dst_hbm.at[pl.ds(row0, RPC)], dbuf)
            cps = [
                pltpu.async_copy(gref.at[sbuf.at[j]],
                                 rows.at[pl.ds(j * ROW, ROW)], gsem)
                for j in range(RPC)
            ]
            for j in range(RPC):
                cps[j].wait()
                pltpu.sync_copy(rows.at[pl.ds(j * ROW, ROW)],
                                acc.at[dbuf.at[j]], add=True)
            return carry

        lax.fori_loop(0, CHUNKS_PER_TILE, body, 0)

    @pl.when(c == 0)
    def _():
        walk(ga_hbm)

    @pl.when(c == 1)
    def _():
        walk(gb_hbm)

    plsc.subcore_barrier()
    for k in range(NPT_CHUNKS):
        pltpu.sync_copy(acc.at[pl.ds(s * NPP + k * ZCH, ZCH)], zbuf)
        pltpu.sync_copy(zbuf, out_hbm.at[pl.ds(c * NP + s * NPP + k * ZCH,
                                               ZCH)])


@functools.partial(
    pl.kernel,
    out_type=[
        jax.ShapeDtypeStruct((NC * GP, HHH), jnp.float32),
        jax.ShapeDtypeStruct((NC * GP, HHH), jnp.float32),
    ],
    mesh=_mesh,
    compiler_params=pltpu.CompilerParams(use_tc_tiling_on_sc=False),
    scratch_types=[
        pltpu.VMEM_SHARED((GP, HHH), jnp.float32),
        pltpu.VMEM_SHARED((GP, HHH), jnp.float32),
        pltpu.VMEM((B_RPC, ROW), jnp.int32),
        pltpu.VMEM((B_RPC * ROW, HHH), jnp.float32),
        pltpu.VMEM((ROW, HHH), jnp.float32),
        pltpu.VMEM((GP, HHH), jnp.float32),
    ],
)
def _pool_kernel(ha_hbm, hb_hbm, batch_hbm, sums_hbm, cnts_hbm, accs, accc,
                 idxb, hbuf, onesb, zb):
    c = lax.axis_index("c")
    s = lax.axis_index("s")

    def zfill(i, carry):
        zb[i] = jnp.zeros((HHH,), jnp.float32)
        return carry

    lax.fori_loop(0, GP, zfill, 0)
    for i in range(ROW):
        onesb[i] = jnp.ones((HHH,), jnp.float32)

    @pl.when(s == 0)
    def _():
        pltpu.sync_copy(zb, accs)
        pltpu.sync_copy(zb, accc)

    plsc.subcore_barrier()

    for q in range(B_PT // B_RPC):
        r0 = s * B_PT + q * B_RPC
        pltpu.sync_copy(batch_hbm.at[pl.ds(r0, B_RPC)], idxb)

        @pl.when(c == 0)
        def _():
            pltpu.sync_copy(ha_hbm.at[pl.ds(r0 * ROW, B_RPC * ROW)], hbuf)

        @pl.when(c == 1)
        def _():
            pltpu.sync_copy(hb_hbm.at[pl.ds(r0 * ROW, B_RPC * ROW)], hbuf)

        for j in range(B_RPC):
            pltpu.sync_copy(hbuf.at[pl.ds(j * ROW, ROW)],
                            accs.at[idxb.at[j]], add=True)
            pltpu.sync_copy(onesb, accc.at[idxb.at[j]], add=True)

    plsc.subcore_barrier()

    @pl.when(s == 0)
    def _():
        pltpu.sync_copy(accs, zb)
        pltpu.sync_copy(zb, sums_hbm.at[pl.ds(c * GP, GP)])
        pltpu.sync_copy(accc, zb)
        pltpu.sync_copy(zb, cnts_hbm.at[pl.ds(c * GP, GP)])


def _tc0_body(x8_ref, cnt_ref, w_ref, ga_ref, gb_ref, dinv_ref):
    deg = cnt_ref[0] + cnt_ref[1] + 1.0
    dinv = lax.rsqrt(deg)
    dinv_ref[...] = dinv
    h2 = jnp.dot(x8_ref[...], w_ref[...], preferred_element_type=jnp.float32)
    ga_ref[...] = h2[:, :ROW] * dinv
    gb_ref[...] = h2[:, ROW:] * dinv


def _tc0(x8, cntp, w0big):
    pspec = pl.BlockSpec((BLK8, ROW), lambda i: (i, 0))
    return pl.pallas_call(
        _tc0_body,
        grid=(GRID,),
        in_specs=[
            pl.BlockSpec((BLK8, 8 * FIN), lambda i: (i, 0)),
            pl.BlockSpec((2, BLK8, ROW), lambda i: (0, i, 0)),
            pl.BlockSpec((8 * FIN, 2 * ROW), lambda i: (0, 0)),
        ],
        out_specs=[pspec, pspec, pspec],
        out_shape=[
            jax.ShapeDtypeStruct((NPK, ROW), jnp.float32),
            jax.ShapeDtypeStruct((NPK, ROW), jnp.float32),
            jax.ShapeDtypeStruct((NPK, ROW), jnp.float32),
        ],
    )(x8, cntp, w0big)


def _tcmid_body(agg_ref, ga_ref, gb_ref, dinv_ref, ba_ref, bb_ref, d_ref,
                goa_ref, gob_ref):
    dinv = dinv_ref[...]
    oa = (agg_ref[0] + ga_ref[...]) * dinv + ba_ref[...]
    ob = (agg_ref[1] + gb_ref[...]) * dinv + bb_ref[...]
    r = jnp.concatenate([jnp.maximum(oa, 0.0), jnp.maximum(ob, 0.0)], axis=1)
    h2 = jnp.dot(r, d_ref[...], preferred_element_type=jnp.float32)
    goa_ref[...] = h2[:, :ROW] * dinv
    gob_ref[...] = h2[:, ROW:] * dinv


def _tcmid(aggp, ga, gb, dinvp, bap, bbp, dmat):
    pspec = pl.BlockSpec((BLK8, ROW), lambda i: (i, 0))
    return pl.pallas_call(
        _tcmid_body,
        grid=(GRID,),
        in_specs=[
            pl.BlockSpec((2, BLK8, ROW), lambda i: (0, i, 0)),
            pspec, pspec, pspec,
            pl.BlockSpec((1, ROW), lambda i: (0, 0)),
            pl.BlockSpec((1, ROW), lambda i: (0, 0)),
            pl.BlockSpec((2 * ROW, 2 * ROW), lambda i: (0, 0)),
        ],
        out_specs=[pspec, pspec],
        out_shape=[
            jax.ShapeDtypeStruct((NPK, ROW), jnp.float32),
            jax.ShapeDtypeStruct((NPK, ROW), jnp.float32),
        ],
    )(aggp, ga, gb, dinvp, bap, bbp, dmat)


def _tcfin_body(agg_ref, ga_ref, gb_ref, dinv_ref, ba_ref, bb_ref,
                ha_ref, hb_ref):
    dinv = dinv_ref[...]
    ha_ref[...] = (agg_ref[0] + ga_ref[...]) * dinv + ba_ref[...]
    hb_ref[...] = (agg_ref[1] + gb_ref[...]) * dinv + bb_ref[...]


def _tcfin(aggp, ga, gb, dinvp, bap, bbp):
    pspec = pl.BlockSpec((BLK8, ROW), lambda i: (i, 0))
    return pl.pallas_call(
        _tcfin_body,
        grid=(GRID,),
        in_specs=[
            pl.BlockSpec((2, BLK8, ROW), lambda i: (0, i, 0)),
            pspec, pspec, pspec,
            pl.BlockSpec((1, ROW), lambda i: (0, 0)),
            pl.BlockSpec((1, ROW), lambda i: (0, 0)),
        ],
        out_specs=[pspec, pspec],
        out_shape=[
            jax.ShapeDtypeStruct((NPK, ROW), jnp.float32),
            jax.ShapeDtypeStruct((NPK, ROW), jnp.float32),
        ],
    )(aggp, ga, gb, dinvp, bap, bbp)


def _tctail_body(sums_ref, cnts_ref, wl_ref, bl_ref, out_ref):
    c1 = jnp.maximum(cnts_ref[0:GG], 1.0)
    pooled = jnp.concatenate(
        [sums_ref[0:GG] / c1, sums_ref[GP:GP + GG] / c1], axis=1)
    out_ref[...] = jnp.dot(
        pooled, wl_ref[...], preferred_element_type=jnp.float32) + bl_ref[...]


def _tctail(sums, cnts, wl, bl):
    return pl.pallas_call(
        _tctail_body,
        grid=(1,),
        in_specs=[
            pl.BlockSpec((NC * GP, HHH), lambda i: (0, 0)),
            pl.BlockSpec((NC * GP, HHH), lambda i: (0, 0)),
            pl.BlockSpec((HH, OUTD), lambda i: (0, 0)),
            pl.BlockSpec((1, OUTD), lambda i: (0, 0)),
        ],
        out_specs=pl.BlockSpec((GG, OUTD), lambda i: (0, 0)),
        out_shape=jax.ShapeDtypeStruct((GG, OUTD), jnp.float32),
    )(sums, cnts, wl, bl)


def _kron8(a):
    return jnp.kron(jnp.eye(8, dtype=jnp.float32), a)


def _dmat(w):
    return jnp.concatenate([
        jnp.concatenate([_kron8(w[:HHH, :HHH]), _kron8(w[:HHH, HHH:])], 1),
        jnp.concatenate([_kron8(w[HHH:, :HHH]), _kron8(w[HHH:, HHH:])], 1),
    ], 0)


def _bpack(b):
    return jnp.tile(b[:HHH], 8).reshape(1, ROW), \
        jnp.tile(b[HHH:], 8).reshape(1, ROW)


def kernel(x, edge_index, batch, W0, b0, W1, b1, W2, b2, Wl, bl):
    npad = E_PAD - EE
    src2 = jnp.concatenate(
        [edge_index[0], jnp.zeros((npad,), jnp.int32)]).reshape(E_ROWS, ROW)
    dst3 = jnp.concatenate(
        [edge_index[1], jnp.full((npad,), NN, jnp.int32)]).reshape(E_ROWS, ROW)
    batchp = jnp.concatenate(
        [batch, jnp.full((NP - NN,), GG, jnp.int32)]).reshape(B_ROWS, ROW)
    x8 = jnp.concatenate(
        [x, jnp.zeros((NP - NN, FIN), jnp.float32)]).reshape(NPK, 8 * FIN)

    w0big = jnp.concatenate(
        [_kron8(W0[:, :HHH]), _kron8(W0[:, HHH:])], axis=1)
    d1 = _dmat(W1)
    d2 = _dmat(W2)
    ba0, bb0 = _bpack(b0)
    ba1, bb1 = _bpack(b1)
    ba2, bb2 = _bpack(b2)

    cntp = _deg_kernel(dst3).reshape(2, NPK, ROW)
    ga, gb, dinvp = _tc0(x8, cntp, w0big)

    agg = _scatter_kernel(ga.reshape(NP, HHH), gb.reshape(NP, HHH),
                          src2, dst3).reshape(2, NPK, ROW)
    ga, gb = _tcmid(agg, ga, gb, dinvp, ba0, bb0, d1)

    agg = _scatter_kernel(ga.reshape(NP, HHH), gb.reshape(NP, HHH),
                          src2, dst3).reshape(2, NPK, ROW)
    ga, gb = _tcmid(agg, ga, gb, dinvp, ba1, bb1, d2)

    agg = _scatter_kernel(ga.reshape(NP, HHH), gb.reshape(NP, HHH),
                          src2, dst3).reshape(2, NPK, ROW)
    ha, hb = _tcfin(agg, ga, gb, dinvp, ba2, bb2)

    sums, cnts = _pool_kernel(ha.reshape(NP, HHH), hb.reshape(NP, HHH),
                              batchp)
    return _tctail(sums, cnts, Wl, bl.reshape(1, OUTD))


# speedup vs baseline: 2.1524x; 1.2022x over previous
import functools

import jax
import jax.numpy as jnp
from jax import lax
from jax.experimental import pallas as pl
from jax.experimental.pallas import tpu as pltpu
from jax.experimental.pallas import tpu_sc as plsc

NN = 100000
EE = 1600000
FIN = 128
HH = 32
HHH = 16
OUTD = 16
GG = 64
GP = 128

NC = 2
NS = 16
ROW = 128
RPC = 4
CHUNK = ROW * RPC

E_ROWS = 12544
E_PAD = E_ROWS * ROW
ROWS_PER_TILE = E_ROWS // NS
CHUNKS_PER_TILE = ROWS_PER_TILE // RPC
DEG_ROWS_PER_W = E_ROWS // (NC * NS)

NP = 100352
NPP = NP // NS
NPK = NP // 8
ZCH = 392
NPT_CHUNKS = NPP // ZCH

B_ROWS = NP // ROW
B_PT = B_ROWS // NS
B_RPC = 7

BLK8 = 784
GRID = NPK // BLK8

_mesh = plsc.VectorSubcoreMesh(
    core_axis_name="c", subcore_axis_name="s", num_cores=NC, num_subcores=NS)


@functools.partial(
    pl.kernel,
    out_type=jax.ShapeDtypeStruct((NC * NP, HHH), jnp.float32),
    mesh=_mesh,
    compiler_params=pltpu.CompilerParams(use_tc_tiling_on_sc=False),
    scratch_types=[
        pltpu.VMEM_SHARED((NP, HHH), jnp.float32),
        pltpu.VMEM((RPC, ROW), jnp.int32),
        pltpu.VMEM((ROW, HHH), jnp.float32),
        pltpu.VMEM((ZCH, HHH), jnp.float32),
    ],
)
def _deg_kernel(dst_hbm, out_hbm, acc, dbuf, onesb, zbuf):
    c = lax.axis_index("c")
    s = lax.axis_index("s")
    w = c * NS + s

    def zfill(i, carry):
        zbuf[i] = jnp.zeros((HHH,), jnp.float32)
        return carry

    lax.fori_loop(0, ZCH, zfill, 0)
    for k in range(NPT_CHUNKS):
        pltpu.sync_copy(zbuf, acc.at[pl.ds(s * NPP + k * ZCH, ZCH)])

    def ofill(i, carry):
        onesb[i] = jnp.ones((HHH,), jnp.float32)
        return carry

    lax.fori_loop(0, ROW, ofill, 0)
    plsc.subcore_barrier()

    def body(i, carry):
        row0 = w * DEG_ROWS_PER_W + i * RPC
        pltpu.sync_copy(dst_hbm.at[pl.ds(row0, RPC)], dbuf)
        for j in range(RPC):
            pltpu.sync_copy(onesb, acc.at[dbuf.at[j]], add=True)
        return carry

    lax.fori_loop(0, DEG_ROWS_PER_W // RPC, body, 0)
    plsc.subcore_barrier()
    for k in range(NPT_CHUNKS):
        pltpu.sync_copy(acc.at[pl.ds(s * NPP + k * ZCH, ZCH)], zbuf)
        pltpu.sync_copy(zbuf, out_hbm.at[pl.ds(c * NP + s * NPP + k * ZCH,
                                               ZCH)])


@functools.partial(
    pl.kernel,
    out_type=jax.ShapeDtypeStruct((NC * NP, HHH), jnp.float32),
    mesh=_mesh,
    compiler_params=pltpu.CompilerParams(use_tc_tiling_on_sc=False),
    scratch_types=[
        pltpu.VMEM_SHARED((NP, HHH), jnp.float32),
        pltpu.VMEM((2, RPC, ROW), jnp.int32),
        pltpu.VMEM((2, RPC, ROW), jnp.int32),
        pltpu.VMEM((2, CHUNK, HHH), jnp.float32),
        pltpu.VMEM((ZCH, HHH), jnp.float32),
        pltpu.SemaphoreType.DMA,
        pltpu.SemaphoreType.DMA,
        pltpu.SemaphoreType.DMA,
    ],
)
def _scatter_kernel(ga_hbm, gb_hbm, src_hbm, dst_hbm, out_hbm, acc, sbuf,
                    dbuf, rows, zbuf, gsem, isem, ssem):
    c = lax.axis_index("c")
    s = lax.axis_index("s")

    def zfill(i, carry):
        zbuf[i] = jnp.zeros((HHH,), jnp.float32)
        return carry

    lax.fori_loop(0, ZCH, zfill, 0)
    for k in range(NPT_CHUNKS):
        pltpu.sync_copy(zbuf, acc.at[pl.ds(s * NPP + k * ZCH, ZCH)])

    plsc.subcore_barrier()

    def walk(gref):
        base = s * ROWS_PER_TILE
        pltpu.async_copy(src_hbm.at[pl.ds(base, RPC)], sbuf.at[0], isem)
        pltpu.async_copy(dst_hbm.at[pl.ds(base, RPC)], dbuf.at[0], isem)

        def body(i, carry):
            for b in range(2):
                g = i * 2 + b
                row0 = base + g * RPC
                pltpu.make_async_copy(src_hbm.at[pl.ds(base, RPC)],
                                      sbuf.at[b], isem).wait()
                pltpu.make_async_copy(dst_hbm.at[pl.ds(base, RPC)],
                                      dbuf.at[b], isem).wait()
                cps = [
                    pltpu.async_copy(gref.at[sbuf.at[b].at[j]],
                                     rows.at[b].at[pl.ds(j * ROW, ROW)],
                                     gsem)
                    for j in range(RPC)
                ]

                @pl.when(g > 0)
                def _():
                    for _j in range(RPC):
                        pltpu.make_async_copy(
                            rows.at[1 - b].at[pl.ds(_j * ROW, ROW)],
                            acc.at[dbuf.at[1 - b].at[_j]], ssem).wait()

                @pl.when(g < CHUNKS_PER_TILE - 1)
                def _():
                    pltpu.async_copy(src_hbm.at[pl.ds(row0 + RPC, RPC)],
                                     sbuf.at[1 - b], isem)
                    pltpu.async_copy(dst_hbm.at[pl.ds(row0 + RPC, RPC)],
                                     dbuf.at[1 - b], isem)

                for j in range(RPC):
                    cps[j].wait()
                    pltpu.async_copy(rows.at[b].at[pl.ds(j * ROW, ROW)],
                                     acc.at[dbuf.at[b].at[j]], ssem,
                                     add=True)
            return carry

        lax.fori_loop(0, CHUNKS_PER_TILE // 2, body, 0)
        for _j in range(RPC):
            pltpu.make_async_copy(
                rows.at[1].at[pl.ds(_j * ROW, ROW)],
                acc.at[dbuf.at[1].at[_j]], ssem).wait()

    @pl.when(c == 0)
    def _():
        walk(ga_hbm)

    @pl.when(c == 1)
    def _():
        walk(gb_hbm)

    plsc.subcore_barrier()
    for k in range(NPT_CHUNKS):
        pltpu.sync_copy(acc.at[pl.ds(s * NPP + k * ZCH, ZCH)], zbuf)
        pltpu.sync_copy(zbuf, out_hbm.at[pl.ds(c * NP + s * NPP + k * ZCH,
                                               ZCH)])


@functools.partial(
    pl.kernel,
    out_type=[
        jax.ShapeDtypeStruct((NC * GP, HHH), jnp.float32),
        jax.ShapeDtypeStruct((NC * GP, HHH), jnp.float32),
    ],
    mesh=_mesh,
    compiler_params=pltpu.CompilerParams(use_tc_tiling_on_sc=False),
    scratch_types=[
        pltpu.VMEM_SHARED((GP, HHH), jnp.float32),
        pltpu.VMEM_SHARED((GP, HHH), jnp.float32),
        pltpu.VMEM((B_RPC, ROW), jnp.int32),
        pltpu.VMEM((B_RPC * ROW, HHH), jnp.float32),
        pltpu.VMEM((ROW, HHH), jnp.float32),
        pltpu.VMEM((GP, HHH), jnp.float32),
    ],
)
def _pool_kernel(ha_hbm, hb_hbm, batch_hbm, sums_hbm, cnts_hbm, accs, accc,
                 idxb, hbuf, onesb, zb):
    c = lax.axis_index("c")
    s = lax.axis_index("s")

    def zfill(i, carry):
        zb[i] = jnp.zeros((HHH,), jnp.float32)
        return carry

    lax.fori_loop(0, GP, zfill, 0)
    for i in range(ROW):
        onesb[i] = jnp.ones((HHH,), jnp.float32)

    @pl.when(s == 0)
    def _():
        pltpu.sync_copy(zb, accs)
        pltpu.sync_copy(zb, accc)

    plsc.subcore_barrier()

    for q in range(B_PT // B_RPC):
        r0 = s * B_PT + q * B_RPC
        pltpu.sync_copy(batch_hbm.at[pl.ds(r0, B_RPC)], idxb)

        @pl.when(c == 0)
        def _():
            pltpu.sync_copy(ha_hbm.at[pl.ds(r0 * ROW, B_RPC * ROW)], hbuf)

        @pl.when(c == 1)
        def _():
            pltpu.sync_copy(hb_hbm.at[pl.ds(r0 * ROW, B_RPC * ROW)], hbuf)

        for j in range(B_RPC):
            pltpu.sync_copy(hbuf.at[pl.ds(j * ROW, ROW)],
                            accs.at[idxb.at[j]], add=True)
            pltpu.sync_copy(onesb, accc.at[idxb.at[j]], add=True)

    plsc.subcore_barrier()

    @pl.when(s == 0)
    def _():
        pltpu.sync_copy(accs, zb)
        pltpu.sync_copy(zb, sums_hbm.at[pl.ds(c * GP, GP)])
        pltpu.sync_copy(accc, zb)
        pltpu.sync_copy(zb, cnts_hbm.at[pl.ds(c * GP, GP)])


def _tc0_body(x8_ref, cnt_ref, w_ref, ga_ref, gb_ref, dinv_ref):
    deg = cnt_ref[0] + cnt_ref[1] + 1.0
    dinv = lax.rsqrt(deg)
    dinv_ref[...] = dinv
    h2 = jnp.dot(x8_ref[...], w_ref[...], preferred_element_type=jnp.float32)
    ga_ref[...] = h2[:, :ROW] * dinv
    gb_ref[...] = h2[:, ROW:] * dinv


def _tc0(x8, cntp, w0big):
    pspec = pl.BlockSpec((BLK8, ROW), lambda i: (i, 0))
    return pl.pallas_call(
        _tc0_body,
        grid=(GRID,),
        in_specs=[
            pl.BlockSpec((BLK8, 8 * FIN), lambda i: (i, 0)),
            pl.BlockSpec((2, BLK8, ROW), lambda i: (0, i, 0)),
            pl.BlockSpec((8 * FIN, 2 * ROW), lambda i: (0, 0)),
        ],
        out_specs=[pspec, pspec, pspec],
        out_shape=[
            jax.ShapeDtypeStruct((NPK, ROW), jnp.float32),
            jax.ShapeDtypeStruct((NPK, ROW), jnp.float32),
            jax.ShapeDtypeStruct((NPK, ROW), jnp.float32),
        ],
    )(x8, cntp, w0big)


def _tcmid_body(agg_ref, ga_ref, gb_ref, dinv_ref, ba_ref, bb_ref, d_ref,
                goa_ref, gob_ref):
    dinv = dinv_ref[...]
    oa = (agg_ref[0] + ga_ref[...]) * dinv + ba_ref[...]
    ob = (agg_ref[1] + gb_ref[...]) * dinv + bb_ref[...]
    r = jnp.concatenate([jnp.maximum(oa, 0.0), jnp.maximum(ob, 0.0)], axis=1)
    h2 = jnp.dot(r, d_ref[...], preferred_element_type=jnp.float32)
    goa_ref[...] = h2[:, :ROW] * dinv
    gob_ref[...] = h2[:, ROW:] * dinv


def _tcmid(aggp, ga, gb, dinvp, bap, bbp, dmat):
    pspec = pl.BlockSpec((BLK8, ROW), lambda i: (i, 0))
    return pl.pallas_call(
        _tcmid_body,
        grid=(GRID,),
        in_specs=[
            pl.BlockSpec((2, BLK8, ROW), lambda i: (0, i, 0)),
            pspec, pspec, pspec,
            pl.BlockSpec((1, ROW), lambda i: (0, 0)),
            pl.BlockSpec((1, ROW), lambda i: (0, 0)),
            pl.BlockSpec((2 * ROW, 2 * ROW), lambda i: (0, 0)),
        ],
        out_specs=[pspec, pspec],
        out_shape=[
            jax.ShapeDtypeStruct((NPK, ROW), jnp.float32),
            jax.ShapeDtypeStruct((NPK, ROW), jnp.float32),
        ],
    )(aggp, ga, gb, dinvp, bap, bbp, dmat)


def _tcfin_body(agg_ref, ga_ref, gb_ref, dinv_ref, ba_ref, bb_ref,
                ha_ref, hb_ref):
    dinv = dinv_ref[...]
    ha_ref[...] = (agg_ref[0] + ga_ref[...]) * dinv + ba_ref[...]
    hb_ref[...] = (agg_ref[1] + gb_ref[...]) * dinv + bb_ref[...]


def _tcfin(aggp, ga, gb, dinvp, bap, bbp):
    pspec = pl.BlockSpec((BLK8, ROW), lambda i: (i, 0))
    return pl.pallas_call(
        _tcfin_body,
        grid=(GRID,),
        in_specs=[
            pl.BlockSpec((2, BLK8, ROW), lambda i: (0, i, 0)),
            pspec, pspec, pspec,
            pl.BlockSpec((1, ROW), lambda i: (0, 0)),
            pl.BlockSpec((1, ROW), lambda i: (0, 0)),
        ],
        out_specs=[pspec, pspec],
        out_shape=[
            jax.ShapeDtypeStruct((NPK, ROW), jnp.float32),
            jax.ShapeDtypeStruct((NPK, ROW), jnp.float32),
        ],
    )(aggp, ga, gb, dinvp, bap, bbp)


def _tctail_body(sums_ref, cnts_ref, wl_ref, bl_ref, out_ref):
    c1 = jnp.maximum(cnts_ref[0:GG], 1.0)
    pooled = jnp.concatenate(
        [sums_ref[0:GG] / c1, sums_ref[GP:GP + GG] / c1], axis=1)
    out_ref[...] = jnp.dot(
        pooled, wl_ref[...], preferred_element_type=jnp.float32) + bl_ref[...]


def _tctail(sums, cnts, wl, bl):
    return pl.pallas_call(
        _tctail_body,
        grid=(1,),
        in_specs=[
            pl.BlockSpec((NC * GP, HHH), lambda i: (0, 0)),
            pl.BlockSpec((NC * GP, HHH), lambda i: (0, 0)),
            pl.BlockSpec((HH, OUTD), lambda i: (0, 0)),
            pl.BlockSpec((1, OUTD), lambda i: (0, 0)),
        ],
        out_specs=pl.BlockSpec((GG, OUTD), lambda i: (0, 0)),
        out_shape=jax.ShapeDtypeStruct((GG, OUTD), jnp.float32),
    )(sums, cnts, wl, bl)


def _kron8(a):
    return jnp.kron(jnp.eye(8, dtype=jnp.float32), a)


def _dmat(w):
    return jnp.concatenate([
        jnp.concatenate([_kron8(w[:HHH, :HHH]), _kron8(w[:HHH, HHH:])], 1),
        jnp.concatenate([_kron8(w[HHH:, :HHH]), _kron8(w[HHH:, HHH:])], 1),
    ], 0)


def _bpack(b):
    return jnp.tile(b[:HHH], 8).reshape(1, ROW), \
        jnp.tile(b[HHH:], 8).reshape(1, ROW)


def kernel(x, edge_index, batch, W0, b0, W1, b1, W2, b2, Wl, bl):
    npad = E_PAD - EE
    src2 = jnp.concatenate(
        [edge_index[0], jnp.zeros((npad,), jnp.int32)]).reshape(E_ROWS, ROW)
    dst3 = jnp.concatenate(
        [edge_index[1], jnp.full((npad,), NN, jnp.int32)]).reshape(E_ROWS, ROW)
    batchp = jnp.concatenate(
        [batch, jnp.full((NP - NN,), GG, jnp.int32)]).reshape(B_ROWS, ROW)
    x8 = jnp.concatenate(
        [x, jnp.zeros((NP - NN, FIN), jnp.float32)]).reshape(NPK, 8 * FIN)

    w0big = jnp.concatenate(
        [_kron8(W0[:, :HHH]), _kron8(W0[:, HHH:])], axis=1)
    d1 = _dmat(W1)
    d2 = _dmat(W2)
    ba0, bb0 = _bpack(b0)
    ba1, bb1 = _bpack(b1)
    ba2, bb2 = _bpack(b2)

    cntp = _deg_kernel(dst3).reshape(2, NPK, ROW)
    ga, gb, dinvp = _tc0(x8, cntp, w0big)

    agg = _scatter_kernel(ga.reshape(NP, HHH), gb.reshape(NP, HHH),
                          src2, dst3).reshape(2, NPK, ROW)
    ga, gb = _tcmid(agg, ga, gb, dinvp, ba0, bb0, d1)

    agg = _scatter_kernel(ga.reshape(NP, HHH), gb.reshape(NP, HHH),
                          src2, dst3).reshape(2, NPK, ROW)
    ga, gb = _tcmid(agg, ga, gb, dinvp, ba1, bb1, d2)

    agg = _scatter_kernel(ga.reshape(NP, HHH), gb.reshape(NP, HHH),
                          src2, dst3).reshape(2, NPK, ROW)
    ha, hb = _tcfin(agg, ga, gb, dinvp, ba2, bb2)

    sums, cnts = _pool_kernel(ha.reshape(NP, HHH), hb.reshape(NP, HHH),
                              batchp)
    return _tctail(sums, cnts, Wl, bl.reshape(1, OUTD))


# speedup vs baseline: 2.2113x; 1.0274x over previous
import functools

import jax
import jax.numpy as jnp
from jax import lax
from jax.experimental import pallas as pl
from jax.experimental.pallas import tpu as pltpu
from jax.experimental.pallas import tpu_sc as plsc

NN = 100000
EE = 1600000
FIN = 128
HH = 32
HHH = 16
OUTD = 16
GG = 64
GP = 128

NC = 2
NS = 16
ROW = 128
RPC = 4
CHUNK = ROW * RPC

E_ROWS = 12544
E_PAD = E_ROWS * ROW
ROWS_PER_TILE = E_ROWS // NS
CHUNKS_PER_TILE = ROWS_PER_TILE // RPC
DEG_ROWS_PER_W = E_ROWS // (NC * NS)

NP = 100352
NPP = NP // NS
NPK = NP // 8
ZCH = 392
NPT_CHUNKS = NPP // ZCH

B_ROWS = NP // ROW
B_PT = B_ROWS // NS
B_RPC = 7

BLK8 = 784
GRID = NPK // BLK8

_mesh = plsc.VectorSubcoreMesh(
    core_axis_name="c", subcore_axis_name="s", num_cores=NC, num_subcores=NS)


@functools.partial(
    pl.kernel,
    out_type=jax.ShapeDtypeStruct((NC * NP, HHH), jnp.float32),
    mesh=_mesh,
    compiler_params=pltpu.CompilerParams(use_tc_tiling_on_sc=False),
    scratch_types=[
        pltpu.VMEM_SHARED((NP, HHH), jnp.float32),
        pltpu.VMEM((2, RPC, ROW), jnp.int32),
        pltpu.VMEM((ROW, HHH), jnp.float32),
        pltpu.VMEM((ZCH, HHH), jnp.float32),
        pltpu.SemaphoreType.DMA,
        pltpu.SemaphoreType.DMA,
    ],
)
def _deg_kernel(dst_hbm, out_hbm, acc, dbuf, onesb, zbuf, isem, ssem):
    c = lax.axis_index("c")
    s = lax.axis_index("s")
    w = c * NS + s
    base = w * DEG_ROWS_PER_W
    nch = DEG_ROWS_PER_W // RPC

    def zfill(i, carry):
        zbuf[i] = jnp.zeros((HHH,), jnp.float32)
        return carry

    lax.fori_loop(0, ZCH, zfill, 0)
    for k in range(NPT_CHUNKS):
        pltpu.sync_copy(zbuf, acc.at[pl.ds(s * NPP + k * ZCH, ZCH)])

    def ofill(i, carry):
        onesb[i] = jnp.ones((HHH,), jnp.float32)
        return carry

    lax.fori_loop(0, ROW, ofill, 0)
    plsc.subcore_barrier()

    pltpu.async_copy(dst_hbm.at[pl.ds(base, RPC)], dbuf.at[0], isem)

    def body(i, carry):
        for b in range(2):
            g = i * 2 + b
            row0 = base + g * RPC
            pltpu.make_async_copy(dst_hbm.at[pl.ds(base, RPC)],
                                  dbuf.at[b], isem).wait()

            @pl.when(g > 0)
            def _():
                for _j in range(RPC):
                    pltpu.make_async_copy(
                        onesb, acc.at[dbuf.at[1 - b].at[_j]], ssem).wait()

            @pl.when(g < nch - 1)
            def _():
                pltpu.async_copy(dst_hbm.at[pl.ds(row0 + RPC, RPC)],
                                 dbuf.at[1 - b], isem)

            for j in range(RPC):
                pltpu.async_copy(onesb, acc.at[dbuf.at[b].at[j]], ssem,
                                 add=True)
        return carry

    lax.fori_loop(0, nch // 2, body, 0)
    for _j in range(RPC):
        pltpu.make_async_copy(onesb, acc.at[dbuf.at[1].at[_j]], ssem).wait()
    plsc.subcore_barrier()
    for k in range(NPT_CHUNKS):
        pltpu.sync_copy(acc.at[pl.ds(s * NPP + k * ZCH, ZCH)], zbuf)
        pltpu.sync_copy(zbuf, out_hbm.at[pl.ds(c * NP + s * NPP + k * ZCH,
                                               ZCH)])


@functools.partial(
    pl.kernel,
    out_type=jax.ShapeDtypeStruct((NC * NP, HHH), jnp.float32),
    mesh=_mesh,
    compiler_params=pltpu.CompilerParams(use_tc_tiling_on_sc=False),
    scratch_types=[
        pltpu.VMEM_SHARED((NP, HHH), jnp.float32),
        pltpu.VMEM((2, RPC, ROW), jnp.int32),
        pltpu.VMEM((2, RPC, ROW), jnp.int32),
        pltpu.VMEM((2, CHUNK, HHH), jnp.float32),
        pltpu.VMEM((ZCH, HHH), jnp.float32),
        pltpu.SemaphoreType.DMA,
        pltpu.SemaphoreType.DMA,
        pltpu.SemaphoreType.DMA,
    ],
)
def _scatter_kernel(ga_hbm, gb_hbm, src_hbm, dst_hbm, out_hbm, acc, sbuf,
                    dbuf, rows, zbuf, gsem, isem, ssem):
    c = lax.axis_index("c")
    s = lax.axis_index("s")

    def zfill(i, carry):
        zbuf[i] = jnp.zeros((HHH,), jnp.float32)
        return carry

    lax.fori_loop(0, ZCH, zfill, 0)
    for k in range(NPT_CHUNKS):
        pltpu.sync_copy(zbuf, acc.at[pl.ds(s * NPP + k * ZCH, ZCH)])

    plsc.subcore_barrier()

    def walk(gref):
        base = s * ROWS_PER_TILE
        pltpu.async_copy(src_hbm.at[pl.ds(base, RPC)], sbuf.at[0], isem)
        pltpu.async_copy(dst_hbm.at[pl.ds(base, RPC)], dbuf.at[0], isem)

        def body(i, carry):
            for b in range(2):
                g = i * 2 + b
                row0 = base + g * RPC
                pltpu.make_async_copy(src_hbm.at[pl.ds(base, RPC)],
                                      sbuf.at[b], isem).wait()
                pltpu.make_async_copy(dst_hbm.at[pl.ds(base, RPC)],
                                      dbuf.at[b], isem).wait()
                cps = [
                    pltpu.async_copy(gref.at[sbuf.at[b].at[j]],
                                     rows.at[b].at[pl.ds(j * ROW, ROW)],
                                     gsem)
                    for j in range(RPC)
                ]

                @pl.when(g > 0)
                def _():
                    for _j in range(RPC):
                        pltpu.make_async_copy(
                            rows.at[1 - b].at[pl.ds(_j * ROW, ROW)],
                            acc.at[dbuf.at[1 - b].at[_j]], ssem).wait()

                @pl.when(g < CHUNKS_PER_TILE - 1)
                def _():
                    pltpu.async_copy(src_hbm.at[pl.ds(row0 + RPC, RPC)],
                                     sbuf.at[1 - b], isem)
                    pltpu.async_copy(dst_hbm.at[pl.ds(row0 + RPC, RPC)],
                                     dbuf.at[1 - b], isem)

                for j in range(RPC):
                    cps[j].wait()
                    pltpu.async_copy(rows.at[b].at[pl.ds(j * ROW, ROW)],
                                     acc.at[dbuf.at[b].at[j]], ssem,
                                     add=True)
            return carry

        lax.fori_loop(0, CHUNKS_PER_TILE // 2, body, 0)
        for _j in range(RPC):
            pltpu.make_async_copy(
                rows.at[1].at[pl.ds(_j * ROW, ROW)],
                acc.at[dbuf.at[1].at[_j]], ssem).wait()

    @pl.when(c == 0)
    def _():
        walk(ga_hbm)

    @pl.when(c == 1)
    def _():
        walk(gb_hbm)

    plsc.subcore_barrier()
    for k in range(NPT_CHUNKS):
        pltpu.sync_copy(acc.at[pl.ds(s * NPP + k * ZCH, ZCH)], zbuf)
        pltpu.sync_copy(zbuf, out_hbm.at[pl.ds(c * NP + s * NPP + k * ZCH,
                                               ZCH)])


@functools.partial(
    pl.kernel,
    out_type=[
        jax.ShapeDtypeStruct((NC * GP, HHH), jnp.float32),
        jax.ShapeDtypeStruct((NC * GP, HHH), jnp.float32),
    ],
    mesh=_mesh,
    compiler_params=pltpu.CompilerParams(use_tc_tiling_on_sc=False),
    scratch_types=[
        pltpu.VMEM_SHARED((GP, HHH), jnp.float32),
        pltpu.VMEM_SHARED((GP, HHH), jnp.float32),
        pltpu.VMEM((B_RPC, ROW), jnp.int32),
        pltpu.VMEM((B_RPC * ROW, HHH), jnp.float32),
        pltpu.VMEM((ROW, HHH), jnp.float32),
        pltpu.VMEM((GP, HHH), jnp.float32),
    ],
)
def _pool_kernel(ha_hbm, hb_hbm, batch_hbm, sums_hbm, cnts_hbm, accs, accc,
                 idxb, hbuf, onesb, zb):
    c = lax.axis_index("c")
    s = lax.axis_index("s")

    def zfill(i, carry):
        zb[i] = jnp.zeros((HHH,), jnp.float32)
        return carry

    lax.fori_loop(0, GP, zfill, 0)
    for i in range(ROW):
        onesb[i] = jnp.ones((HHH,), jnp.float32)

    @pl.when(s == 0)
    def _():
        pltpu.sync_copy(zb, accs)
        pltpu.sync_copy(zb, accc)

    plsc.subcore_barrier()

    for q in range(B_PT // B_RPC):
        r0 = s * B_PT + q * B_RPC
        pltpu.sync_copy(batch_hbm.at[pl.ds(r0, B_RPC)], idxb)

        @pl.when(c == 0)
        def _():
            pltpu.sync_copy(ha_hbm.at[pl.ds(r0 * ROW, B_RPC * ROW)], hbuf)

        @pl.when(c == 1)
        def _():
            pltpu.sync_copy(hb_hbm.at[pl.ds(r0 * ROW, B_RPC * ROW)], hbuf)

        for j in range(B_RPC):
            pltpu.sync_copy(hbuf.at[pl.ds(j * ROW, ROW)],
                            accs.at[idxb.at[j]], add=True)
            pltpu.sync_copy(onesb, accc.at[idxb.at[j]], add=True)

    plsc.subcore_barrier()

    @pl.when(s == 0)
    def _():
        pltpu.sync_copy(accs, zb)
        pltpu.sync_copy(zb, sums_hbm.at[pl.ds(c * GP, GP)])
        pltpu.sync_copy(accc, zb)
        pltpu.sync_copy(zb, cnts_hbm.at[pl.ds(c * GP, GP)])


def _tcmm_body(x8_ref, w_ref, h2_ref):
    h2_ref[...] = jnp.dot(x8_ref[...], w_ref[...],
                          preferred_element_type=jnp.float32)


def _tcmm(x8, w0big):
    return pl.pallas_call(
        _tcmm_body,
        grid=(GRID,),
        in_specs=[
            pl.BlockSpec((BLK8, 8 * FIN), lambda i: (i, 0)),
            pl.BlockSpec((8 * FIN, 2 * ROW), lambda i: (0, 0)),
        ],
        out_specs=pl.BlockSpec((BLK8, 2 * ROW), lambda i: (i, 0)),
        out_shape=jax.ShapeDtypeStruct((NPK, 2 * ROW), jnp.float32),
    )(x8, w0big)


def _tc0_body(h2_ref, cnt_ref, ga_ref, gb_ref, dinv_ref):
    deg = cnt_ref[0] + cnt_ref[1] + 1.0
    dinv = lax.rsqrt(deg)
    dinv_ref[...] = dinv
    h2 = h2_ref[...]
    ga_ref[...] = h2[:, :ROW] * dinv
    gb_ref[...] = h2[:, ROW:] * dinv


def _tc0(h2, cntp):
    pspec = pl.BlockSpec((BLK8, ROW), lambda i: (i, 0))
    return pl.pallas_call(
        _tc0_body,
        grid=(GRID,),
        in_specs=[
            pl.BlockSpec((BLK8, 2 * ROW), lambda i: (i, 0)),
            pl.BlockSpec((2, BLK8, ROW), lambda i: (0, i, 0)),
        ],
        out_specs=[pspec, pspec, pspec],
        out_shape=[
            jax.ShapeDtypeStruct((NPK, ROW), jnp.float32),
            jax.ShapeDtypeStruct((NPK, ROW), jnp.float32),
            jax.ShapeDtypeStruct((NPK, ROW), jnp.float32),
        ],
    )(h2, cntp)


def _tcmid_body(agg_ref, ga_ref, gb_ref, dinv_ref, ba_ref, bb_ref, d_ref,
                goa_ref, gob_ref):
    dinv = dinv_ref[...]
    oa = (agg_ref[0] + ga_ref[...]) * dinv + ba_ref[...]
    ob = (agg_ref[1] + gb_ref[...]) * dinv + bb_ref[...]
    r = jnp.concatenate([jnp.maximum(oa, 0.0), jnp.maximum(ob, 0.0)], axis=1)
    h2 = jnp.dot(r, d_ref[...], preferred_element_type=jnp.float32)
    goa_ref[...] = h2[:, :ROW] * dinv
    gob_ref[...] = h2[:, ROW:] * dinv


def _tcmid(aggp, ga, gb, dinvp, bap, bbp, dmat):
    pspec = pl.BlockSpec((BLK8, ROW), lambda i: (i, 0))
    return pl.pallas_call(
        _tcmid_body,
        grid=(GRID,),
        in_specs=[
            pl.BlockSpec((2, BLK8, ROW), lambda i: (0, i, 0)),
            pspec, pspec, pspec,
            pl.BlockSpec((1, ROW), lambda i: (0, 0)),
            pl.BlockSpec((1, ROW), lambda i: (0, 0)),
            pl.BlockSpec((2 * ROW, 2 * ROW), lambda i: (0, 0)),
        ],
        out_specs=[pspec, pspec],
        out_shape=[
            jax.ShapeDtypeStruct((NPK, ROW), jnp.float32),
            jax.ShapeDtypeStruct((NPK, ROW), jnp.float32),
        ],
    )(aggp, ga, gb, dinvp, bap, bbp, dmat)


def _tcfin_body(agg_ref, ga_ref, gb_ref, dinv_ref, ba_ref, bb_ref,
                ha_ref, hb_ref):
    dinv = dinv_ref[...]
    ha_ref[...] = (agg_ref[0] + ga_ref[...]) * dinv + ba_ref[...]
    hb_ref[...] = (agg_ref[1] + gb_ref[...]) * dinv + bb_ref[...]


def _tcfin(aggp, ga, gb, dinvp, bap, bbp):
    pspec = pl.BlockSpec((BLK8, ROW), lambda i: (i, 0))
    return pl.pallas_call(
        _tcfin_body,
        grid=(GRID,),
        in_specs=[
            pl.BlockSpec((2, BLK8, ROW), lambda i: (0, i, 0)),
            pspec, pspec, pspec,
            pl.BlockSpec((1, ROW), lambda i: (0, 0)),
            pl.BlockSpec((1, ROW), lambda i: (0, 0)),
        ],
        out_specs=[pspec, pspec],
        out_shape=[
            jax.ShapeDtypeStruct((NPK, ROW), jnp.float32),
            jax.ShapeDtypeStruct((NPK, ROW), jnp.float32),
        ],
    )(aggp, ga, gb, dinvp, bap, bbp)


def _tctail_body(sums_ref, cnts_ref, wl_ref, bl_ref, out_ref):
    c1 = jnp.maximum(cnts_ref[0:GG], 1.0)
    pooled = jnp.concatenate(
        [sums_ref[0:GG] / c1, sums_ref[GP:GP + GG] / c1], axis=1)
    out_ref[...] = jnp.dot(
        pooled, wl_ref[...], preferred_element_type=jnp.float32) + bl_ref[...]


def _tctail(sums, cnts, wl, bl):
    return pl.pallas_call(
        _tctail_body,
        grid=(1,),
        in_specs=[
            pl.BlockSpec((NC * GP, HHH), lambda i: (0, 0)),
            pl.BlockSpec((NC * GP, HHH), lambda i: (0, 0)),
            pl.BlockSpec((HH, OUTD), lambda i: (0, 0)),
            pl.BlockSpec((1, OUTD), lambda i: (0, 0)),
        ],
        out_specs=pl.BlockSpec((GG, OUTD), lambda i: (0, 0)),
        out_shape=jax.ShapeDtypeStruct((GG, OUTD), jnp.float32),
    )(sums, cnts, wl, bl)


def _kron8(a):
    return jnp.kron(jnp.eye(8, dtype=jnp.float32), a)


def _dmat(w):
    return jnp.concatenate([
        jnp.concatenate([_kron8(w[:HHH, :HHH]), _kron8(w[:HHH, HHH:])], 1),
        jnp.concatenate([_kron8(w[HHH:, :HHH]), _kron8(w[HHH:, HHH:])], 1),
    ], 0)


def _bpack(b):
    return jnp.tile(b[:HHH], 8).reshape(1, ROW), \
        jnp.tile(b[HHH:], 8).reshape(1, ROW)


def kernel(x, edge_index, batch, W0, b0, W1, b1, W2, b2, Wl, bl):
    npad = E_PAD - EE
    src2 = jnp.concatenate(
        [edge_index[0], jnp.zeros((npad,), jnp.int32)]).reshape(E_ROWS, ROW)
    dst3 = jnp.concatenate(
        [edge_index[1], jnp.full((npad,), NN, jnp.int32)]).reshape(E_ROWS, ROW)
    batchp = jnp.concatenate(
        [batch, jnp.full((NP - NN,), GG, jnp.int32)]).reshape(B_ROWS, ROW)
    x8 = jnp.concatenate(
        [x, jnp.zeros((NP - NN, FIN), jnp.float32)]).reshape(NPK, 8 * FIN)

    w0big = jnp.concatenate(
        [_kron8(W0[:, :HHH]), _kron8(W0[:, HHH:])], axis=1)
    d1 = _dmat(W1)
    d2 = _dmat(W2)
    ba0, bb0 = _bpack(b0)
    ba1, bb1 = _bpack(b1)
    ba2, bb2 = _bpack(b2)

    h2 = _tcmm(x8, w0big)
    cntp = _deg_kernel(dst3).reshape(2, NPK, ROW)
    ga, gb, dinvp = _tc0(h2, cntp)

    agg = _scatter_kernel(ga.reshape(NP, HHH), gb.reshape(NP, HHH),
                          src2, dst3).reshape(2, NPK, ROW)
    ga, gb = _tcmid(agg, ga, gb, dinvp, ba0, bb0, d1)

    agg = _scatter_kernel(ga.reshape(NP, HHH), gb.reshape(NP, HHH),
                          src2, dst3).reshape(2, NPK, ROW)
    ga, gb = _tcmid(agg, ga, gb, dinvp, ba1, bb1, d2)

    agg = _scatter_kernel(ga.reshape(NP, HHH), gb.reshape(NP, HHH),
                          src2, dst3).reshape(2, NPK, ROW)
    ha, hb = _tcfin(agg, ga, gb, dinvp, ba2, bb2)

    sums, cnts = _pool_kernel(ha.reshape(NP, HHH), hb.reshape(NP, HHH),
                              batchp)
    return _tctail(sums, cnts, Wl, bl.reshape(1, OUTD))
